# baseline (speedup 1.0000x reference)
import jax
import jax.numpy as jnp
from jax import lax
from jax.experimental import pallas as pl
from jax.experimental.pallas import tpu as pltpu
from jax.experimental.pallas import tpu_sc as plsc

D = 64
B = 16384
L = 16
NC, NS = 2, 16
NW = NC * NS
BPW = B // NW
CHUNK = 128
NCHUNK = BPW // CHUNK
CBLK = 65536


def _score_body(w_ref, t_ref, o_ref):
    x = t_ref[...]
    w = w_ref[...]
    y = jnp.sum(x * w, axis=0)
    o_ref[...] = y.reshape(CBLK // 128, 128)


def _scores(table_t, w_col):
    n = table_t.shape[1]
    grid = (n + CBLK - 1) // CBLK
    return pl.pallas_call(
        _score_body,
        grid=(grid,),
        in_specs=[
            pl.BlockSpec((D, 1), lambda i: (0, 0)),
            pl.BlockSpec((D, CBLK), lambda i: (0, i)),
        ],
        out_specs=pl.BlockSpec((CBLK // 128, 128), lambda i: (i, 0)),
        out_shape=jax.ShapeDtypeStruct((grid * (CBLK // 128), 128),
                                       jnp.float32),
    )(w_col, table_t)


def _lookup_body(user_idx, cat_idx, su, sc, b_vec, out_hbm,
                 uidx_v, cidx_v, pu_v, pc_v, su_buf, sc_buf, out_v, b_v,
                 sem_u, sem_c):
    wid = lax.axis_index("s") * NC + lax.axis_index("c")
    base = wid * BPW

    pltpu.sync_copy(user_idx.at[pl.ds(base, BPW)], uidx_v)
    pltpu.sync_copy(cat_idx.at[pl.ds(base, BPW)], cidx_v)
    pltpu.sync_copy(b_vec, b_v)

    def prep_body(g, carry):
        gb = g * L
        uv = uidx_v[pl.ds(gb, L)]
        cv = cidx_v[pl.ds(gb, L)]
        pu_v[pl.ds(gb, L)] = lax.shift_right_logical(uv, 7)
        pc_v[pl.ds(gb, L)] = lax.shift_right_logical(cv, 7)
        return carry

    lax.fori_loop(0, BPW // L, prep_body, 0)

    lanes = lax.iota(jnp.int32, L)
    bv = b_v[...]

    for c in range(NCHUNK):
        hu = pltpu.async_copy(su.at[pu_v.at[pl.ds(c * CHUNK, CHUNK)]],
                              su_buf, sem_u)
        hc = pltpu.async_copy(sc.at[pc_v.at[pl.ds(c * CHUNK, CHUNK)]],
                              sc_buf, sem_c)
        hu.wait()
        hc.wait()

        def grp_body(g, carry):
            gb = g * L
            uv = uidx_v[pl.ds(c * CHUNK + gb, L)]
            cv = cidx_v[pl.ds(c * CHUNK + gb, L)]
            rows = gb + lanes
            vu = plsc.load_gather(su_buf, [rows, uv & 127])
            vc = plsc.load_gather(sc_buf, [rows, cv & 127])
            x = vu + vc + bv
            out_v[pl.ds(c * CHUNK + gb, L)] = 1.0 / (1.0 + jnp.exp(-x))
            return carry

        lax.fori_loop(0, CHUNK // L, grp_body, 0)

    pltpu.sync_copy(out_v, out_hbm.at[pl.ds(base, BPW)])


def kernel(user, category, user_table, category_table, fc_w, fc_b):
    ut_t = user_table.T
    ct_t = category_table.T
    wu_col = fc_w[0, :D].reshape(D, 1)
    wc_col = fc_w[0, D:].reshape(D, 1)
    b_vec = jnp.broadcast_to(fc_b.reshape(1), (L,))

    s_u = _scores(ut_t, wu_col)
    s_c = _scores(ct_t, wc_col)

    run = pl.kernel(
        _lookup_body,
        out_type=jax.ShapeDtypeStruct((B,), jnp.float32),
        mesh=plsc.VectorSubcoreMesh(core_axis_name="c", subcore_axis_name="s"),
        compiler_params=pltpu.CompilerParams(needs_layout_passes=False),
        scratch_types=[
            pltpu.VMEM((BPW,), jnp.int32),
            pltpu.VMEM((BPW,), jnp.int32),
            pltpu.VMEM((BPW,), jnp.int32),
            pltpu.VMEM((BPW,), jnp.int32),
            pltpu.VMEM((CHUNK, 128), jnp.float32),
            pltpu.VMEM((CHUNK, 128), jnp.float32),
            pltpu.VMEM((BPW,), jnp.float32),
            pltpu.VMEM((L,), jnp.float32),
            pltpu.SemaphoreType.DMA,
            pltpu.SemaphoreType.DMA,
        ],
    )
    return run(user, category, s_u, s_c, b_vec)


# speedup vs baseline: 1.0327x; 1.0327x over previous
import jax
import jax.numpy as jnp
from jax import lax
from jax.experimental import pallas as pl
from jax.experimental.pallas import tpu as pltpu
from jax.experimental.pallas import tpu_sc as plsc

D = 64
B = 16384
L = 16
NC, NS = 2, 16
NW = NC * NS
BPW = B // NW
CHUNK = 128
NCHUNK = BPW // CHUNK
CBLK = 32768


def _score_body(w_ref, t_ref, o_ref):
    x = t_ref[...]
    w = w_ref[...]
    y = jnp.sum(x * w, axis=0)
    o_ref[...] = y.reshape(CBLK // 128, 128)


def _scores(table_t, w_col):
    n = table_t.shape[1]
    grid = (n + CBLK - 1) // CBLK
    return pl.pallas_call(
        _score_body,
        grid=(grid,),
        in_specs=[
            pl.BlockSpec((D, 1), lambda i: (0, 0)),
            pl.BlockSpec((D, CBLK), lambda i: (0, i)),
        ],
        out_specs=pl.BlockSpec((CBLK // 128, 128), lambda i: (i, 0)),
        out_shape=jax.ShapeDtypeStruct((grid * (CBLK // 128), 128),
                                       jnp.float32),
    )(w_col, table_t)


def _lookup_body(user_idx, cat_idx, su, sc, b_vec, out_hbm,
                 uidx_v, cidx_v, pu_v, pc_v, su_buf, sc_buf, out_v, b_v,
                 sem_u, sem_c):
    wid = lax.axis_index("s") * NC + lax.axis_index("c")
    base = wid * BPW

    pltpu.sync_copy(user_idx.at[pl.ds(base, BPW)], uidx_v)
    pltpu.sync_copy(cat_idx.at[pl.ds(base, BPW)], cidx_v)
    pltpu.sync_copy(b_vec, b_v)

    def prep_body(g, carry):
        gb = g * L
        uv = uidx_v[pl.ds(gb, L)]
        cv = cidx_v[pl.ds(gb, L)]
        pu_v[pl.ds(gb, L)] = lax.shift_right_logical(uv, 7)
        pc_v[pl.ds(gb, L)] = lax.shift_right_logical(cv, 7)
        return carry

    lax.fori_loop(0, BPW // L, prep_body, 0)

    lanes = lax.iota(jnp.int32, L)
    bv = b_v[...]

    for c in range(NCHUNK):
        hu = pltpu.async_copy(su.at[pu_v.at[pl.ds(c * CHUNK, CHUNK)]],
                              su_buf, sem_u)
        hc = pltpu.async_copy(sc.at[pc_v.at[pl.ds(c * CHUNK, CHUNK)]],
                              sc_buf, sem_c)
        hu.wait()
        hc.wait()

        def grp_body(g, carry):
            gb = g * L
            uv = uidx_v[pl.ds(c * CHUNK + gb, L)]
            cv = cidx_v[pl.ds(c * CHUNK + gb, L)]
            rows = gb + lanes
            vu = plsc.load_gather(su_buf, [rows, uv & 127])
            vc = plsc.load_gather(sc_buf, [rows, cv & 127])
            x = vu + vc + bv
            out_v[pl.ds(c * CHUNK + gb, L)] = 1.0 / (1.0 + jnp.exp(-x))
            return carry

        lax.fori_loop(0, CHUNK // L, grp_body, 0)

    pltpu.sync_copy(out_v, out_hbm.at[pl.ds(base, BPW)])


def kernel(user, category, user_table, category_table, fc_w, fc_b):
    ut_t = user_table.T
    ct_t = category_table.T
    wu_col = fc_w[0, :D].reshape(D, 1)
    wc_col = fc_w[0, D:].reshape(D, 1)
    b_vec = jnp.broadcast_to(fc_b.reshape(1), (L,))

    s_u = _scores(ut_t, wu_col)
    s_c = _scores(ct_t, wc_col)

    run = pl.kernel(
        _lookup_body,
        out_type=jax.ShapeDtypeStruct((B,), jnp.float32),
        mesh=plsc.VectorSubcoreMesh(core_axis_name="c", subcore_axis_name="s"),
        compiler_params=pltpu.CompilerParams(needs_layout_passes=False),
        scratch_types=[
            pltpu.VMEM((BPW,), jnp.int32),
            pltpu.VMEM((BPW,), jnp.int32),
            pltpu.VMEM((BPW,), jnp.int32),
            pltpu.VMEM((BPW,), jnp.int32),
            pltpu.VMEM((CHUNK, 128), jnp.float32),
            pltpu.VMEM((CHUNK, 128), jnp.float32),
            pltpu.VMEM((BPW,), jnp.float32),
            pltpu.VMEM((L,), jnp.float32),
            pltpu.SemaphoreType.DMA,
            pltpu.SemaphoreType.DMA,
        ],
    )
    return run(user, category, s_u, s_c, b_vec)


# speedup vs baseline: 1.0405x; 1.0076x over previous
import jax
import jax.numpy as jnp
from jax import lax
from jax.experimental import pallas as pl
from jax.experimental.pallas import tpu as pltpu
from jax.experimental.pallas import tpu_sc as plsc

D = 64
B = 16384
L = 16
NC, NS = 2, 16
NW = NC * NS
BPW = B // NW
CHUNK = 128
NCHUNK = BPW // CHUNK
CBLK = 32768


def _score_body(w_ref, t_ref, o_ref):
    x = t_ref[...]
    w = w_ref[...]
    y = jnp.sum(x * w, axis=0)
    o_ref[...] = y.reshape(CBLK // 128, 128)


def _scores(table_t, w_col):
    n = table_t.shape[1]
    grid = (n + CBLK - 1) // CBLK
    return pl.pallas_call(
        _score_body,
        grid=(grid,),
        in_specs=[
            pl.BlockSpec((D, 1), lambda i: (0, 0)),
            pl.BlockSpec((D, CBLK), lambda i: (0, i)),
        ],
        out_specs=pl.BlockSpec((CBLK // 128, 128), lambda i: (i, 0)),
        out_shape=jax.ShapeDtypeStruct((grid * (CBLK // 128), 128),
                                       jnp.float32),
    )(w_col, table_t)


def _lookup_body(user_idx, cat_idx, su, sc, b_vec, out_hbm,
                 uidx_v, cidx_v, pu_v, pc_v, su_buf, sc_buf, out_v, b_v,
                 sem_u, sem_c):
    wid = lax.axis_index("s") * NC + lax.axis_index("c")
    base = wid * BPW

    pltpu.sync_copy(user_idx.at[pl.ds(base, BPW)], uidx_v)
    pltpu.sync_copy(cat_idx.at[pl.ds(base, BPW)], cidx_v)
    pltpu.sync_copy(b_vec, b_v)

    def prep_body(g, carry):
        gb = g * L
        uv = uidx_v[pl.ds(gb, L)]
        cv = cidx_v[pl.ds(gb, L)]
        pu_v[pl.ds(gb, L)] = lax.shift_right_logical(uv, 7)
        pc_v[pl.ds(gb, L)] = lax.shift_right_logical(cv, 7)
        return carry

    lax.fori_loop(0, BPW // L, prep_body, 0)

    lanes = lax.iota(jnp.int32, L)
    bv = b_v[...]

    def fire(c):
        s = c % 2
        hu = pltpu.async_copy(su.at[pu_v.at[pl.ds(c * CHUNK, CHUNK)]],
                              su_buf.at[s], sem_u)
        hc = pltpu.async_copy(sc.at[pc_v.at[pl.ds(c * CHUNK, CHUNK)]],
                              sc_buf.at[s], sem_c)
        return hu, hc

    handles = {0: fire(0)}
    for c in range(NCHUNK):
        if c + 1 < NCHUNK:
            handles[c + 1] = fire(c + 1)
        hu, hc = handles.pop(c)
        hu.wait()
        hc.wait()
        s = c % 2

        def grp_body(g, carry):
            gb = g * L
            uv = uidx_v[pl.ds(c * CHUNK + gb, L)]
            cv = cidx_v[pl.ds(c * CHUNK + gb, L)]
            rows = gb + lanes
            vu = plsc.load_gather(su_buf.at[s], [rows, uv & 127])
            vc = plsc.load_gather(sc_buf.at[s], [rows, cv & 127])
            x = vu + vc + bv
            out_v[pl.ds(c * CHUNK + gb, L)] = 1.0 / (1.0 + jnp.exp(-x))
            return carry

        lax.fori_loop(0, CHUNK // L, grp_body, 0)

    pltpu.sync_copy(out_v, out_hbm.at[pl.ds(base, BPW)])


def kernel(user, category, user_table, category_table, fc_w, fc_b):
    ut_t = user_table.T
    ct_t = category_table.T
    wu_col = fc_w[0, :D].reshape(D, 1)
    wc_col = fc_w[0, D:].reshape(D, 1)
    b_vec = jnp.broadcast_to(fc_b.reshape(1), (L,))

    s_u = _scores(ut_t, wu_col)
    s_c = _scores(ct_t, wc_col)

    run = pl.kernel(
        _lookup_body,
        out_type=jax.ShapeDtypeStruct((B,), jnp.float32),
        mesh=plsc.VectorSubcoreMesh(core_axis_name="c", subcore_axis_name="s"),
        compiler_params=pltpu.CompilerParams(needs_layout_passes=False),
        scratch_types=[
            pltpu.VMEM((BPW,), jnp.int32),
            pltpu.VMEM((BPW,), jnp.int32),
            pltpu.VMEM((BPW,), jnp.int32),
            pltpu.VMEM((BPW,), jnp.int32),
            pltpu.VMEM((2, CHUNK, 128), jnp.float32),
            pltpu.VMEM((2, CHUNK, 128), jnp.float32),
            pltpu.VMEM((BPW,), jnp.float32),
            pltpu.VMEM((L,), jnp.float32),
            pltpu.SemaphoreType.DMA,
            pltpu.SemaphoreType.DMA,
        ],
    )
    return run(user, category, s_u, s_c, b_vec)
